# baseline (speedup 1.0000x reference)
import functools

import jax
import jax.numpy as jnp
from jax import lax
from jax.experimental import pallas as pl
from jax.experimental.pallas import tpu as pltpu
from jax.experimental.pallas import tpu_sc as plsc

N = 10000
E = 320000
D_IN = 128
EMB = 128
H1 = 32
H2 = 64
BN_EPS = 1e-5

NC = 2
NS = 16
NW = NC * NS
CHUNK = 128
CHW = 80
E_PAD = NW * CHW * CHUNK
N_PAD = 10112
RPS = N_PAD // NS
KB = 1


@functools.lru_cache(maxsize=None)
def _make_seg_sum(F):
    mesh = plsc.VectorSubcoreMesh(core_axis_name="c", subcore_axis_name="s")

    @functools.partial(
        pl.kernel,
        mesh=mesh,
        compiler_params=pltpu.CompilerParams(use_tc_tiling_on_sc=False),
        out_type=jax.ShapeDtypeStruct((NC, N_PAD, F), jnp.float32),
        scratch_types=[
            pltpu.VMEM((CHW, CHUNK), jnp.int32),
            pltpu.VMEM((CHW, CHUNK), jnp.int32),
            [pltpu.VMEM((CHUNK, F), jnp.float32) for _ in range(KB)],
            pltpu.VMEM_SHARED((N_PAD, F), jnp.float32),
            pltpu.SemaphoreType.DMA,
            pltpu.SemaphoreType.DMA,
        ],
    )
    def seg_sum(y_hbm, src_hbm, dst_hbm, zeros_hbm, out_hbm,
                src_v, dst_v, rows, acc, sem_g, sem_s):
        c = lax.axis_index("c")
        s = lax.axis_index("s")
        w = c * NS + s

        pltpu.sync_copy(zeros_hbm, acc.at[pl.ds(s * RPS, RPS)])
        pltpu.sync_copy(src_hbm.at[w], src_v)
        pltpu.sync_copy(dst_hbm.at[w], dst_v)
        plsc.subcore_barrier()

        def body(j, carry):
            pltpu.async_copy(y_hbm.at[src_v.at[j]], rows[0], sem_g).wait()
            pltpu.async_copy(rows[0], acc.at[dst_v.at[j]], sem_s,
                             add=True).wait()
            return carry

        lax.fori_loop(0, CHW, body, 0)

        plsc.subcore_barrier()
        pltpu.sync_copy(acc.at[pl.ds(s * RPS, RPS)],
                        out_hbm.at[c, pl.ds(s * RPS, RPS)])

    return seg_sum


def _qdot(a, b):
    return jnp.dot(a.astype(jnp.bfloat16), b.astype(jnp.bfloat16),
                   preferred_element_type=jnp.float32)


def _tc_pre_body(x_ref, w_ref, y_ref):
    y_ref[...] = _qdot(x_ref[...], w_ref[...])


def _tc_mid_body(y_ref, a_ref, ba_ref, g_ref, be_ref, wb_ref, bb_ref, wn_ref,
                 u_ref):
    z = y_ref[...] + a_ref[0, :N] + a_ref[1, :N] + ba_ref[...]
    mu = jnp.mean(z, axis=0, keepdims=True)
    zc = z - mu
    var = jnp.mean(zc * zc, axis=0, keepdims=True)
    h = zc * lax.rsqrt(var + BN_EPS) * g_ref[...] + be_ref[...]
    h = jnp.maximum(h, 0.0)
    h = _qdot(h, wb_ref[...]) + bb_ref[...]
    h = jnp.maximum(h, 0.0)
    u_ref[...] = _qdot(h, wn_ref[...])


def _tc_fin_body(u_ref, a_ref, ba_ref, g_ref, be_ref, wb_ref, bb_ref,
                 h_ref, p_ref):
    z = u_ref[...] + a_ref[0, :N] + a_ref[1, :N] + ba_ref[...]
    mu = jnp.mean(z, axis=0, keepdims=True)
    zc = z - mu
    var = jnp.mean(zc * zc, axis=0, keepdims=True)
    h = zc * lax.rsqrt(var + BN_EPS) * g_ref[...] + be_ref[...]
    h = jnp.maximum(h, 0.0)
    h = _qdot(h, wb_ref[...]) + bb_ref[...]
    h = jnp.maximum(h, 0.0)
    h_ref[...] = h
    p_ref[...] = jnp.mean(h, axis=0, keepdims=True)


def kernel(x, edge_index, W1a, b1a, g1, be1, W1b, b1b,
           W2a, b2a, g2, be2, W2b, b2b):
    pad = E_PAD - E
    src_p = jnp.concatenate(
        [edge_index[0], jnp.zeros((pad,), jnp.int32)]
    ).reshape(NW, CHW, CHUNK)
    dst_p = jnp.concatenate(
        [edge_index[1], jnp.full((pad,), N_PAD - 1, jnp.int32)]
    ).reshape(NW, CHW, CHUNK)
    z32 = jnp.zeros((RPS, H1), jnp.float32)
    z64 = jnp.zeros((RPS, H2), jnp.float32)

    y1 = pl.pallas_call(
        _tc_pre_body,
        out_shape=jax.ShapeDtypeStruct((N, H1), jnp.float32),
    )(x, W1a)

    a1 = _make_seg_sum(H1)(y1, src_p, dst_p, z32)

    u = pl.pallas_call(
        _tc_mid_body,
        out_shape=jax.ShapeDtypeStruct((N, H2), jnp.float32),
    )(y1, a1, b1a.reshape(1, H1), g1.reshape(1, H1), be1.reshape(1, H1),
      W1b, b1b.reshape(1, H2), W2a)

    a2 = _make_seg_sum(H2)(u, src_p, dst_p, z64)

    h, pooled = pl.pallas_call(
        _tc_fin_body,
        out_shape=[
            jax.ShapeDtypeStruct((N, EMB), jnp.float32),
            jax.ShapeDtypeStruct((1, EMB), jnp.float32),
        ],
    )(u, a2, b2a.reshape(1, H2), g2.reshape(1, H2), be2.reshape(1, H2),
      W2b, b2b.reshape(1, EMB))

    return (h, pooled)


# speedup vs baseline: 1.3304x; 1.3304x over previous
import functools

import jax
import jax.numpy as jnp
from jax import lax
from jax.experimental import pallas as pl
from jax.experimental.pallas import tpu as pltpu
from jax.experimental.pallas import tpu_sc as plsc

N = 10000
E = 320000
D_IN = 128
EMB = 128
H1 = 32
H2 = 64
BN_EPS = 1e-5

NC = 2
NS = 16
NW = NC * NS
CHUNK = 128
CHW = (E // CHUNK + NW - 1) // NW
E_PAD = NW * CHW * CHUNK
N_PAD = 10112
RPS = N_PAD // NS
KB = 1


@functools.lru_cache(maxsize=None)
def _make_seg_sum(F):
    mesh = plsc.VectorSubcoreMesh(core_axis_name="c", subcore_axis_name="s")

    @functools.partial(
        pl.kernel,
        mesh=mesh,
        compiler_params=pltpu.CompilerParams(use_tc_tiling_on_sc=False),
        out_type=jax.ShapeDtypeStruct((NC, N_PAD, F), jnp.float32),
        scratch_types=[
            pltpu.VMEM((CHW, CHUNK), jnp.int32),
            pltpu.VMEM((CHW, CHUNK), jnp.int32),
            pltpu.VMEM((CHUNK, F), jnp.float32),
            pltpu.VMEM_SHARED((N_PAD, F), jnp.float32),
            pltpu.SemaphoreType.DMA,
        ],
    )
    def seg_sum(y_hbm, src_hbm, dst_hbm, zeros_hbm, out_hbm,
                src_v, dst_v, rows_v, acc, sem):
        c = lax.axis_index("c")
        s = lax.axis_index("s")
        w = c * NS + s

        pltpu.sync_copy(zeros_hbm, acc.at[pl.ds(s * RPS, RPS)])
        pltpu.sync_copy(src_hbm.at[w], src_v)
        pltpu.sync_copy(dst_hbm.at[w], dst_v)
        plsc.subcore_barrier()

        def body(j, carry):
            pltpu.async_copy(y_hbm.at[src_v.at[j]], rows_v, sem).wait()
            pltpu.sync_copy(rows_v, acc.at[dst_v.at[j]], add=True)
            return carry

        lax.fori_loop(0, CHW, body, 0)

        plsc.subcore_barrier()
        pltpu.sync_copy(acc.at[pl.ds(s * RPS, RPS)],
                        out_hbm.at[c, pl.ds(s * RPS, RPS)])

    return seg_sum


def _qdot(a, b):
    return jnp.dot(a.astype(jnp.bfloat16), b.astype(jnp.bfloat16),
                   preferred_element_type=jnp.float32)


def _tc_pre_body(x_ref, w_ref, y_ref):
    y_ref[...] = _qdot(x_ref[...], w_ref[...])


def _tc_mid_body(y_ref, a_ref, ba_ref, g_ref, be_ref, wb_ref, bb_ref, wn_ref,
                 u_ref):
    z = y_ref[...] + a_ref[0, :N] + a_ref[1, :N] + ba_ref[...]
    mu = jnp.mean(z, axis=0, keepdims=True)
    zc = z - mu
    var = jnp.mean(zc * zc, axis=0, keepdims=True)
    h = zc * lax.rsqrt(var + BN_EPS) * g_ref[...] + be_ref[...]
    h = jnp.maximum(h, 0.0)
    h = _qdot(h, wb_ref[...]) + bb_ref[...]
    h = jnp.maximum(h, 0.0)
    u_ref[...] = _qdot(h, wn_ref[...])


def _tc_fin_body(u_ref, a_ref, ba_ref, g_ref, be_ref, wb_ref, bb_ref,
                 h_ref, p_ref):
    z = u_ref[...] + a_ref[0, :N] + a_ref[1, :N] + ba_ref[...]
    mu = jnp.mean(z, axis=0, keepdims=True)
    zc = z - mu
    var = jnp.mean(zc * zc, axis=0, keepdims=True)
    h = zc * lax.rsqrt(var + BN_EPS) * g_ref[...] + be_ref[...]
    h = jnp.maximum(h, 0.0)
    h = _qdot(h, wb_ref[...]) + bb_ref[...]
    h = jnp.maximum(h, 0.0)
    h_ref[...] = h
    p_ref[...] = jnp.mean(h, axis=0, keepdims=True)


def kernel(x, edge_index, W1a, b1a, g1, be1, W1b, b1b,
           W2a, b2a, g2, be2, W2b, b2b):
    pad = E_PAD - E
    src_p = jnp.concatenate(
        [edge_index[0], jnp.zeros((pad,), jnp.int32)]
    ).reshape(NW, CHW, CHUNK)
    dst_p = jnp.concatenate(
        [edge_index[1], jnp.full((pad,), N_PAD - 1, jnp.int32)]
    ).reshape(NW, CHW, CHUNK)
    z32 = jnp.zeros((RPS, H1), jnp.float32)
    z64 = jnp.zeros((RPS, H2), jnp.float32)

    y1 = pl.pallas_call(
        _tc_pre_body,
        out_shape=jax.ShapeDtypeStruct((N, H1), jnp.float32),
    )(x, W1a)

    a1 = _make_seg_sum(H1)(y1, src_p, dst_p, z32)

    u = pl.pallas_call(
        _tc_mid_body,
        out_shape=jax.ShapeDtypeStruct((N, H2), jnp.float32),
    )(y1, a1, b1a.reshape(1, H1), g1.reshape(1, H1), be1.reshape(1, H1),
      W1b, b1b.reshape(1, H2), W2a)

    a2 = _make_seg_sum(H2)(u, src_p, dst_p, z64)

    h, pooled = pl.pallas_call(
        _tc_fin_body,
        out_shape=[
            jax.ShapeDtypeStruct((N, EMB), jnp.float32),
            jax.ShapeDtypeStruct((1, EMB), jnp.float32),
        ],
    )(u, a2, b2a.reshape(1, H2), g2.reshape(1, H2), be2.reshape(1, H2),
      W2b, b2b.reshape(1, EMB))

    return (h, pooled)
